# baseline (speedup 1.0000x reference)
import jax
import jax.numpy as jnp
from jax.experimental import pallas as pl

N = 262144
PAD = 0


def kernel(ast_node_types, ast_node_major_types, ast_node_minor_types, ast_node_nr_children, ast_node_child_ltr_position, ast_node_child_rtl_position, id_leaf_node_indices, id_leaf_identifier_idx, prim_leaf_node_indices, prim_leaf_primitive_type, mod_leaf_node_indices, mod_leaf_modifier, identifiers_encodings, type_emb, major_emb, minor_emb, nrc_emb, pos_emb, prim_emb, mod_emb, W_id, b_id, W_prim, b_prim, W_mod, b_mod, W_wo, b_wo):
    n_id = id_leaf_node_indices.shape[0]
    n_pt = prim_leaf_node_indices.shape[0]
    n_mod = mod_leaf_node_indices.shape[0]

    types_e = jnp.take(type_emb, ast_node_types, axis=0)
    major_e = jnp.take(major_emb, ast_node_major_types, axis=0)
    minor_e = jnp.take(minor_emb, ast_node_minor_types, axis=0)
    major_minor = jnp.concatenate([major_e, minor_e], axis=-1)
    types_e = jnp.where((ast_node_minor_types == PAD)[:, None], types_e, major_minor)
    nrc_e = jnp.take(nrc_emb, ast_node_nr_children, axis=0)
    pos_e = jnp.take(pos_emb, ast_node_child_ltr_position, axis=0) + jnp.take(pos_emb, ast_node_child_rtl_position, axis=0)
    wo_ipm = jnp.concatenate([types_e, nrc_e, pos_e], axis=-1)
    base = jax.nn.relu(wo_ipm @ W_wo + b_wo)
    id_enc = jnp.take(identifiers_encodings, id_leaf_identifier_idx, axis=0)
    id_nodes = jax.nn.relu(jnp.concatenate([id_enc, jnp.take(wo_ipm, id_leaf_node_indices, axis=0)], axis=-1) @ W_id + b_id)
    prim_e = jnp.take(prim_emb, prim_leaf_primitive_type, axis=0)
    prim_nodes = jax.nn.relu(jnp.concatenate([prim_e, jnp.take(wo_ipm, prim_leaf_node_indices, axis=0)], axis=-1) @ W_prim + b_prim)
    mod_e = jnp.take(mod_emb, mod_leaf_modifier, axis=0)
    mod_nodes = jax.nn.relu(jnp.concatenate([mod_e, jnp.take(wo_ipm, mod_leaf_node_indices, axis=0)], axis=-1) @ W_mod + b_mod)

    key_id = 1 + jnp.arange(n_id, dtype=jnp.int32)
    key_prim = 1 + n_id + jnp.arange(n_pt, dtype=jnp.int32)
    key_mod = 1 + n_id + n_pt + jnp.arange(n_mod, dtype=jnp.int32)
    aux = jnp.zeros((N,), dtype=jnp.int32)
    aux = aux.at[id_leaf_node_indices].max(key_id)
    aux = aux.at[prim_leaf_node_indices].max(key_prim)
    aux = aux.at[mod_leaf_node_indices].max(key_mod)
    win_id = aux[id_leaf_node_indices] == key_id
    win_prim = aux[prim_leaf_node_indices] == key_prim
    win_mod = aux[mod_leaf_node_indices] == key_mod

    out = base
    out = out.at[jnp.where(win_id, id_leaf_node_indices, N)].set(id_nodes, mode="drop")
    out = out.at[jnp.where(win_prim, prim_leaf_node_indices, N)].set(prim_nodes, mode="drop")
    out = out.at[jnp.where(win_mod, mod_leaf_node_indices, N)].set(mod_nodes, mode="drop")
    return out


# speedup vs baseline: 14.1626x; 14.1626x over previous
import functools

import jax
import jax.numpy as jnp
from jax import lax
from jax.experimental import pallas as pl
from jax.experimental.pallas import tpu as pltpu
from jax.experimental.pallas import tpu_sc as plsc

N = 262144
P_ID = 65536
P_PT = 32768
P_MOD = 32768
P = P_ID + P_PT + P_MOD
C = P + N
D = 128
PAD = 0

TILE = 1024
NC = 2
NS = 16
NW = NC * NS

_f32 = jnp.float32
_i32 = jnp.int32
_u8 = jnp.uint8


def _oh(idx_col, v, dtype=_f32):
    rows = idx_col.shape[0]
    io = lax.broadcasted_iota(_i32, (rows, v), 1)
    return (idx_col == io).astype(dtype)


def _unpack(w0, w1):
    typ = w0 & 255
    maj = (w0 >> 8) & 31
    mnr = (w0 >> 13) & 63
    nrc = (w0 >> 19) & 31
    ltr = w1 & 63
    rtl = (w1 >> 6) & 63
    return typ, maj, mnr, nrc, ltr, rtl


def _base_body(w0_ref, w1_ref, t1_ref, t2_ref, t3_ref, t4_ref, t5_ref, b_ref,
               out_ref):
    typ, maj, mnr, nrc, ltr, rtl = _unpack(w0_ref[...], w1_ref[...])
    first = jnp.where(
        mnr == PAD,
        jnp.dot(_oh(typ, 200), t1_ref[...], preferred_element_type=_f32),
        jnp.dot(_oh(maj, 32), t2_ref[...], preferred_element_type=_f32)
        + jnp.dot(_oh(mnr, 64), t3_ref[...], preferred_element_type=_f32),
    )
    acc = (first
           + jnp.dot(_oh(nrc, 32), t4_ref[...], preferred_element_type=_f32)
           + jnp.dot(_oh(ltr, 64) + _oh(rtl, 64), t5_ref[...],
                     preferred_element_type=_f32)
           + b_ref[...])
    out_ref[...] = jnp.maximum(acc, 0.0)


def _leaf_body(comb_in_ref, w0_ref, w1_ref, idenc_ref, featpm_ref, wida_ref,
               t1_ref, t2_ref, t3_ref, t4_ref, t5_ref, ft_ref, b_ref,
               out_ref):
    del comb_in_ref
    i = pl.program_id(0)
    typ, maj, mnr, nrc, ltr, rtl = _unpack(w0_ref[...], w1_ref[...])
    first = jnp.where(
        mnr == PAD,
        jnp.dot(_oh(typ, 200), t1_ref[0], preferred_element_type=_f32),
        jnp.dot(_oh(maj, 32), t2_ref[0], preferred_element_type=_f32)
        + jnp.dot(_oh(mnr, 64), t3_ref[0], preferred_element_type=_f32),
    )
    acc = (first
           + jnp.dot(_oh(nrc, 32), t4_ref[0], preferred_element_type=_f32)
           + jnp.dot(_oh(ltr, 64) + _oh(rtl, 64), t5_ref[0],
                     preferred_element_type=_f32)
           + b_ref[0])

    @pl.when(i < P_ID // TILE)
    def _id_region():
        feat = jnp.dot(idenc_ref[...], wida_ref[...],
                       preferred_element_type=_f32)
        out_ref[...] = jnp.maximum(acc + feat, 0.0)

    @pl.when(i >= P_ID // TILE)
    def _pm_region():
        feat = jnp.dot(_oh(featpm_ref[...], 16), ft_ref[0],
                       preferred_element_type=_f32)
        out_ref[...] = jnp.maximum(acc + feat, 0.0)


def _sc_gather(w0_hbm, w1_hbm, lnodes_hbm, ident_hbm, ididx_hbm,
               w0_out, w1_out, idrows_out,
               idxa_v, idxi_v, b0, b1, rbuf, sem):
    wid = lax.axis_index("s") * NC + lax.axis_index("c")
    a0 = wid * (P // NW)
    pltpu.sync_copy(lnodes_hbm.at[pl.ds(a0, 4096)], idxa_v)

    def _ga(k, _):
        sl = pl.ds(k * 128, 128)
        pltpu.async_copy(w0_hbm.at[idxa_v.at[sl]], b0.at[sl], sem).wait()
        pltpu.async_copy(w1_hbm.at[idxa_v.at[sl]], b1.at[sl], sem).wait()
        return 0
    lax.fori_loop(0, 32, _ga, 0)
    pltpu.sync_copy(b0, w0_out.at[pl.ds(a0, 4096)])
    pltpu.sync_copy(b1, w1_out.at[pl.ds(a0, 4096)])
    i0 = wid * (P_ID // NW)
    pltpu.sync_copy(ididx_hbm.at[pl.ds(i0, 2048)], idxi_v)
    for oc in range(4):
        def _gi(k, _, oc=oc):
            pltpu.async_copy(
                ident_hbm.at[idxi_v.at[pl.ds(oc * 512 + k * 128, 128)]],
                rbuf.at[pl.ds(k * 128, 128)], sem).wait()
            return 0
        lax.fori_loop(0, 4, _gi, 0)
        pltpu.sync_copy(rbuf, idrows_out.at[pl.ds(i0 + oc * 512, 512)])


def _sc_emit(src_hbm, comb_hbm, out_hbm, srcv, rbuf, sem):
    wid = lax.axis_index("s") * NC + lax.axis_index("c")
    n0 = wid * (N // NW)

    def _chunk(ch, _):
        base = n0 + ch * 512
        pltpu.sync_copy(src_hbm.at[pl.ds(base, 512)], srcv)

        def _g(k, _):
            pltpu.async_copy(
                comb_hbm.at[srcv.at[pl.ds(k * 128, 128)]],
                rbuf.at[pl.ds(k * 128, 128)], sem).wait()
            return 0
        lax.fori_loop(0, 4, _g, 0)
        pltpu.sync_copy(rbuf, out_hbm.at[pl.ds(base, 512)])
        return 0
    lax.fori_loop(0, N // NW // 512, _chunk, 0)


def _sc_mesh():
    return plsc.VectorSubcoreMesh(core_axis_name="c", subcore_axis_name="s",
                                  num_cores=NC, num_subcores=NS)


def _gather_call(*args):
    return pl.kernel(
        _sc_gather,
        out_type=(jax.ShapeDtypeStruct((P,), _i32),
                  jax.ShapeDtypeStruct((P,), _i32),
                  jax.ShapeDtypeStruct((P_ID, D), _f32)),
        scratch_types=[
            pltpu.VMEM((4096,), _i32),
            pltpu.VMEM((2048,), _i32),
            pltpu.VMEM((4096,), _i32),
            pltpu.VMEM((4096,), _i32),
            pltpu.VMEM((512, D), _f32),
            pltpu.SemaphoreType.DMA,
        ],
        mesh=_sc_mesh(),
    )(*args)


def _emit_call(*args):
    return pl.kernel(
        _sc_emit,
        out_type=jax.ShapeDtypeStruct((N, D), _f32),
        scratch_types=[
            pltpu.VMEM((512,), _i32),
            pltpu.VMEM((512, D), _f32),
            pltpu.SemaphoreType.DMA,
        ],
        mesh=_sc_mesh(),
    )(*args)


def _fold(table, w_part):
    return jnp.dot(table, w_part, preferred_element_type=_f32)


def _fold_stage(type_emb, major_emb, minor_emb, nrc_emb, pos_emb, w):
    a, b, c = w[0:128], w[128:160], w[160:192]
    t1 = _fold(type_emb, a)
    t2 = jnp.pad(_fold(major_emb, a[0:85]), ((0, 2), (0, 0)))
    t3 = jnp.pad(_fold(minor_emb, a[85:128]), ((0, 4), (0, 0)))
    t4 = _fold(nrc_emb, b)
    t5 = _fold(pos_emb, c)
    return t1, t2, t3, t4, t5


def kernel(ast_node_types, ast_node_major_types, ast_node_minor_types,
           ast_node_nr_children, ast_node_child_ltr_position,
           ast_node_child_rtl_position, id_leaf_node_indices,
           id_leaf_identifier_idx, prim_leaf_node_indices,
           prim_leaf_primitive_type, mod_leaf_node_indices, mod_leaf_modifier,
           identifiers_encodings, type_emb, major_emb, minor_emb, nrc_emb,
           pos_emb, prim_emb, mod_emb, W_id, b_id, W_prim, b_prim, W_mod,
           b_mod, W_wo, b_wo):
    ii = lambda x: x.astype(_i32)
    w0 = (ii(ast_node_types) | (ii(ast_node_major_types) << 8)
          | (ii(ast_node_minor_types) << 13) | (ii(ast_node_nr_children) << 19))
    w1 = ii(ast_node_child_ltr_position) | (ii(ast_node_child_rtl_position) << 6)
    leaf_nodes = jnp.concatenate([ii(id_leaf_node_indices),
                                  ii(prim_leaf_node_indices),
                                  ii(mod_leaf_node_indices)])
    featpm = jnp.concatenate([ii(prim_leaf_primitive_type),
                              ii(mod_leaf_modifier)]).reshape(-1, 1)

    keys = jnp.arange(1, P + 1, dtype=_i32)
    aux = jnp.zeros((N,), _i32).at[leaf_nodes].max(keys)
    src = jnp.where(aux > 0, aux - 1, jnp.arange(N, dtype=_i32) + P)

    tw = _fold_stage(type_emb, major_emb, minor_emb, nrc_emb, pos_emb, W_wo)
    tid = _fold_stage(type_emb, major_emb, minor_emb, nrc_emb, pos_emb,
                      W_id[D:])
    tpr = _fold_stage(type_emb, major_emb, minor_emb, nrc_emb, pos_emb,
                      W_prim[64:])
    tmo = _fold_stage(type_emb, major_emb, minor_emb, nrc_emb, pos_emb,
                      W_mod[64:])
    stk = [jnp.stack([tid[j], tpr[j], tmo[j]]) for j in range(5)]
    ft_stk = jnp.stack([jnp.zeros((16, D), _f32),
                        _fold(prim_emb, W_prim[0:64]),
                        _fold(mod_emb, W_mod[0:64])])
    b_stk = jnp.stack([b_id.reshape(1, D), b_prim.reshape(1, D),
                       b_mod.reshape(1, D)])
    w_ida = W_id[0:D]

    full = lambda s: pl.BlockSpec(s, lambda i: (0,) * len(s))
    combined0 = pl.pallas_call(
        _base_body,
        grid=(N // TILE,),
        in_specs=[
            pl.BlockSpec((TILE, 1), lambda i: (i, 0)),
            pl.BlockSpec((TILE, 1), lambda i: (i, 0)),
            full((200, D)), full((32, D)), full((64, D)), full((32, D)),
            full((64, D)), full((1, D)),
        ],
        out_specs=pl.BlockSpec((TILE, D), lambda i: (i + P // TILE, 0)),
        out_shape=jax.ShapeDtypeStruct((C, D), _f32),
    )(w0.reshape(N, 1), w1.reshape(N, 1), *tw, b_wo.reshape(1, D))

    w0_leaf, w1_leaf, idrows = _gather_call(w0, w1, leaf_nodes,
                                            identifiers_encodings,
                                            ii(id_leaf_identifier_idx))

    nid = P_ID // TILE
    r_of = lambda i: jnp.where(i < nid, 0,
                               jnp.where(i < nid + P_PT // TILE, 1, 2))
    combined = pl.pallas_call(
        _leaf_body,
        grid=(P // TILE,),
        in_specs=[
            pl.BlockSpec(memory_space=pl.ANY),
            pl.BlockSpec((TILE, 1), lambda i: (i, 0)),
            pl.BlockSpec((TILE, 1), lambda i: (i, 0)),
            pl.BlockSpec((TILE, D), lambda i: (jnp.minimum(i, nid - 1), 0)),
            pl.BlockSpec((TILE, 1),
                         lambda i: (jnp.clip(i - nid, 0, nid - 1), 0)),
            full((D, D)),
            pl.BlockSpec((1, 200, D), lambda i: (r_of(i), 0, 0)),
            pl.BlockSpec((1, 32, D), lambda i: (r_of(i), 0, 0)),
            pl.BlockSpec((1, 64, D), lambda i: (r_of(i), 0, 0)),
            pl.BlockSpec((1, 32, D), lambda i: (r_of(i), 0, 0)),
            pl.BlockSpec((1, 64, D), lambda i: (r_of(i), 0, 0)),
            pl.BlockSpec((1, 16, D), lambda i: (r_of(i), 0, 0)),
            pl.BlockSpec((1, 1, D), lambda i: (r_of(i), 0, 0)),
        ],
        out_specs=pl.BlockSpec((TILE, D), lambda i: (i, 0)),
        out_shape=jax.ShapeDtypeStruct((C, D), _f32),
        input_output_aliases={0: 0},
    )(combined0, w0_leaf.reshape(P, 1), w1_leaf.reshape(P, 1), idrows,
      featpm, w_ida, *stk, ft_stk, b_stk)

    return _emit_call(src, combined)


# speedup vs baseline: 15.4223x; 1.0889x over previous
import functools

import jax
import jax.numpy as jnp
from jax import lax
from jax.experimental import pallas as pl
from jax.experimental.pallas import tpu as pltpu
from jax.experimental.pallas import tpu_sc as plsc

N = 262144
P_ID = 65536
P_PT = 32768
P_MOD = 32768
P = P_ID + P_PT + P_MOD
C = P + N
D = 128
PAD = 0

TILE = 1024
NC = 2
NS = 16
NW = NC * NS

_f32 = jnp.float32
_i32 = jnp.int32
_u8 = jnp.uint8


def _oh(idx_col, v, dtype=_f32):
    rows = idx_col.shape[0]
    io = lax.broadcasted_iota(_i32, (rows, v), 1)
    return (idx_col == io).astype(dtype)


def _unpack(w0, w1):
    typ = w0 & 255
    maj = (w0 >> 8) & 31
    mnr = (w0 >> 13) & 63
    nrc = (w0 >> 19) & 31
    ltr = w1 & 63
    rtl = (w1 >> 6) & 63
    return typ, maj, mnr, nrc, ltr, rtl


def _base_body(w0_ref, w1_ref, t1_ref, t2_ref, t3_ref, t4_ref, t5_ref, b_ref,
               out_ref):
    typ, maj, mnr, nrc, ltr, rtl = _unpack(w0_ref[...], w1_ref[...])
    first = jnp.where(
        mnr == PAD,
        jnp.dot(_oh(typ, 200), t1_ref[...], preferred_element_type=_f32),
        jnp.dot(_oh(maj, 32), t2_ref[...], preferred_element_type=_f32)
        + jnp.dot(_oh(mnr, 64), t3_ref[...], preferred_element_type=_f32),
    )
    acc = (first
           + jnp.dot(_oh(nrc, 32), t4_ref[...], preferred_element_type=_f32)
           + jnp.dot(_oh(ltr, 64) + _oh(rtl, 64), t5_ref[...],
                     preferred_element_type=_f32)
           + b_ref[...])
    out_ref[...] = jnp.maximum(acc, 0.0)


def _leaf_body(comb_in_ref, w0_ref, w1_ref, idenc_ref, featpm_ref, wida_ref,
               t1_ref, t2_ref, t3_ref, t4_ref, t5_ref, ft_ref, b_ref,
               out_ref):
    del comb_in_ref
    i = pl.program_id(0)
    typ, maj, mnr, nrc, ltr, rtl = _unpack(w0_ref[...], w1_ref[...])
    first = jnp.where(
        mnr == PAD,
        jnp.dot(_oh(typ, 200), t1_ref[0], preferred_element_type=_f32),
        jnp.dot(_oh(maj, 32), t2_ref[0], preferred_element_type=_f32)
        + jnp.dot(_oh(mnr, 64), t3_ref[0], preferred_element_type=_f32),
    )
    acc = (first
           + jnp.dot(_oh(nrc, 32), t4_ref[0], preferred_element_type=_f32)
           + jnp.dot(_oh(ltr, 64) + _oh(rtl, 64), t5_ref[0],
                     preferred_element_type=_f32)
           + b_ref[0])

    @pl.when(i < P_ID // TILE)
    def _id_region():
        feat = jnp.dot(idenc_ref[...], wida_ref[...],
                       preferred_element_type=_f32)
        out_ref[...] = jnp.maximum(acc + feat, 0.0)

    @pl.when(i >= P_ID // TILE)
    def _pm_region():
        feat = jnp.dot(_oh(featpm_ref[...], 16), ft_ref[0],
                       preferred_element_type=_f32)
        out_ref[...] = jnp.maximum(acc + feat, 0.0)


def _sc_gather(w0_hbm, w1_hbm, lnodes_hbm, ident_hbm, ididx_hbm,
               w0_out, w1_out, idrows_out, src_out,
               idxa_v, idxi_v, b0, b1, rbuf, aux_v, sbuf, srcb, tmp16, sem):
    wid = lax.axis_index("s") * NC + lax.axis_index("c")
    a0 = wid * (P // NW)
    pltpu.sync_copy(lnodes_hbm.at[pl.ds(a0, 4096)], idxa_v)

    def _ga(k, _):
        sl = pl.ds(k * 128, 128)
        pltpu.async_copy(w0_hbm.at[idxa_v.at[sl]], b0.at[sl], sem).wait()
        pltpu.async_copy(w1_hbm.at[idxa_v.at[sl]], b1.at[sl], sem).wait()
        return 0
    lax.fori_loop(0, 32, _ga, 0)
    pltpu.sync_copy(b0, w0_out.at[pl.ds(a0, 4096)])
    pltpu.sync_copy(b1, w1_out.at[pl.ds(a0, 4096)])
    i0 = wid * (P_ID // NW)
    pltpu.sync_copy(ididx_hbm.at[pl.ds(i0, 2048)], idxi_v)
    for oc in range(4):
        def _gi(k, _, oc=oc):
            pltpu.async_copy(
                ident_hbm.at[idxi_v.at[pl.ds(oc * 512 + k * 128, 128)]],
                rbuf.at[pl.ds(k * 128, 128)], sem).wait()
            return 0
        lax.fori_loop(0, 4, _gi, 0)
        pltpu.sync_copy(rbuf, idrows_out.at[pl.ds(i0 + oc * 512, 512)])

    R = N // NW
    lo = wid * R
    zeros16 = jnp.zeros((16,), _i32)

    def _z(j, _):
        aux_v[pl.ds(j * 16, 16)] = zeros16
        return 0
    lax.fori_loop(0, R // 16, _z, 0)
    iota16 = lax.iota(_i32, 16)
    nxtidx = jnp.minimum(iota16 + 1, 15)

    def _outer(cb, _):
        pltpu.sync_copy(lnodes_hbm.at[pl.ds(cb * 4096, 4096)], sbuf)

        def _inner(j, _):
            v = sbuf[pl.ds(j * 16, 16)]
            m = (v >= lo) & (v < lo + R)

            @pl.when(jnp.any(m))
            def _hit():
                key = cb * 4096 + j * 16 + 1 + iota16
                c = jnp.where(m, ((v - lo) << 18) | key, -1)
                sc, _unused = plsc.sort_key_val(c, c)
                tmp16[...] = sc
                nxt = plsc.load_gather(tmp16, [nxtidx])
                islast = ((sc >> 18) != (nxt >> 18)) | (iota16 == 15)
                upd = islast & (sc >= 0)
                addr = jnp.maximum(sc >> 18, 0)
                val = sc & 0x3FFFF
                old = plsc.load_gather(aux_v, [addr], mask=upd)
                plsc.store_scatter(aux_v, [addr], jnp.maximum(old, val),
                                   mask=upd)
            return 0
        lax.fori_loop(0, 256, _inner, 0)
        return 0
    lax.fori_loop(0, P // 4096, _outer, 0)

    def _src(q, _):
        def _sv(j, _):
            k = aux_v[pl.ds(q * 2048 + j * 16, 16)]
            node = lo + q * 2048 + j * 16 + iota16
            srcb[pl.ds(j * 16, 16)] = jnp.where(k > 0, k - 1, node + P)
            return 0
        lax.fori_loop(0, 128, _sv, 0)
        pltpu.sync_copy(srcb, src_out.at[pl.ds(lo + q * 2048, 2048)])
        return 0
    lax.fori_loop(0, R // 2048, _src, 0)


def _sc_emit(src_hbm, comb_hbm, out_hbm, srcv, rbuf, sem):
    wid = lax.axis_index("s") * NC + lax.axis_index("c")
    n0 = wid * (N // NW)

    def _chunk(ch, _):
        base = n0 + ch * 512
        pltpu.sync_copy(src_hbm.at[pl.ds(base, 512)], srcv)

        def _g(k, _):
            pltpu.async_copy(
                comb_hbm.at[srcv.at[pl.ds(k * 128, 128)]],
                rbuf.at[pl.ds(k * 128, 128)], sem).wait()
            return 0
        lax.fori_loop(0, 4, _g, 0)
        pltpu.sync_copy(rbuf, out_hbm.at[pl.ds(base, 512)])
        return 0
    lax.fori_loop(0, N // NW // 512, _chunk, 0)


def _sc_mesh():
    return plsc.VectorSubcoreMesh(core_axis_name="c", subcore_axis_name="s",
                                  num_cores=NC, num_subcores=NS)


def _gather_call(*args):
    return pl.kernel(
        _sc_gather,
        out_type=(jax.ShapeDtypeStruct((P,), _i32),
                  jax.ShapeDtypeStruct((P,), _i32),
                  jax.ShapeDtypeStruct((P_ID, D), _f32),
                  jax.ShapeDtypeStruct((N,), _i32)),
        scratch_types=[
            pltpu.VMEM((4096,), _i32),
            pltpu.VMEM((2048,), _i32),
            pltpu.VMEM((4096,), _i32),
            pltpu.VMEM((4096,), _i32),
            pltpu.VMEM((512, D), _f32),
            pltpu.VMEM((N // NW,), _i32),
            pltpu.VMEM((4096,), _i32),
            pltpu.VMEM((2048,), _i32),
            pltpu.VMEM((16,), _i32),
            pltpu.SemaphoreType.DMA,
        ],
        mesh=_sc_mesh(),
        compiler_params=pltpu.CompilerParams(needs_layout_passes=False),
    )(*args)


def _emit_call(*args):
    return pl.kernel(
        _sc_emit,
        out_type=jax.ShapeDtypeStruct((N, D), _f32),
        scratch_types=[
            pltpu.VMEM((512,), _i32),
            pltpu.VMEM((512, D), _f32),
            pltpu.SemaphoreType.DMA,
        ],
        mesh=_sc_mesh(),
        compiler_params=pltpu.CompilerParams(needs_layout_passes=False),
    )(*args)


def _fold(table, w_part):
    return jnp.dot(table, w_part, preferred_element_type=_f32)


def _fold_stage(type_emb, major_emb, minor_emb, nrc_emb, pos_emb, w):
    a, b, c = w[0:128], w[128:160], w[160:192]
    t1 = _fold(type_emb, a)
    t2 = jnp.pad(_fold(major_emb, a[0:85]), ((0, 2), (0, 0)))
    t3 = jnp.pad(_fold(minor_emb, a[85:128]), ((0, 4), (0, 0)))
    t4 = _fold(nrc_emb, b)
    t5 = _fold(pos_emb, c)
    return t1, t2, t3, t4, t5


def kernel(ast_node_types, ast_node_major_types, ast_node_minor_types,
           ast_node_nr_children, ast_node_child_ltr_position,
           ast_node_child_rtl_position, id_leaf_node_indices,
           id_leaf_identifier_idx, prim_leaf_node_indices,
           prim_leaf_primitive_type, mod_leaf_node_indices, mod_leaf_modifier,
           identifiers_encodings, type_emb, major_emb, minor_emb, nrc_emb,
           pos_emb, prim_emb, mod_emb, W_id, b_id, W_prim, b_prim, W_mod,
           b_mod, W_wo, b_wo):
    ii = lambda x: x.astype(_i32)
    w0 = (ii(ast_node_types) | (ii(ast_node_major_types) << 8)
          | (ii(ast_node_minor_types) << 13) | (ii(ast_node_nr_children) << 19))
    w1 = ii(ast_node_child_ltr_position) | (ii(ast_node_child_rtl_position) << 6)
    leaf_nodes = jnp.concatenate([ii(id_leaf_node_indices),
                                  ii(prim_leaf_node_indices),
                                  ii(mod_leaf_node_indices)])
    featpm = jnp.concatenate([ii(prim_leaf_primitive_type),
                              ii(mod_leaf_modifier)]).reshape(-1, 1)

    tw = _fold_stage(type_emb, major_emb, minor_emb, nrc_emb, pos_emb, W_wo)
    tid = _fold_stage(type_emb, major_emb, minor_emb, nrc_emb, pos_emb,
                      W_id[D:])
    tpr = _fold_stage(type_emb, major_emb, minor_emb, nrc_emb, pos_emb,
                      W_prim[64:])
    tmo = _fold_stage(type_emb, major_emb, minor_emb, nrc_emb, pos_emb,
                      W_mod[64:])
    stk = [jnp.stack([tid[j], tpr[j], tmo[j]]) for j in range(5)]
    ft_stk = jnp.stack([jnp.zeros((16, D), _f32),
                        _fold(prim_emb, W_prim[0:64]),
                        _fold(mod_emb, W_mod[0:64])])
    b_stk = jnp.stack([b_id.reshape(1, D), b_prim.reshape(1, D),
                       b_mod.reshape(1, D)])
    w_ida = W_id[0:D]

    full = lambda s: pl.BlockSpec(s, lambda i: (0,) * len(s))
    combined0 = pl.pallas_call(
        _base_body,
        grid=(N // TILE,),
        in_specs=[
            pl.BlockSpec((TILE, 1), lambda i: (i, 0)),
            pl.BlockSpec((TILE, 1), lambda i: (i, 0)),
            full((200, D)), full((32, D)), full((64, D)), full((32, D)),
            full((64, D)), full((1, D)),
        ],
        out_specs=pl.BlockSpec((TILE, D), lambda i: (i + P // TILE, 0)),
        out_shape=jax.ShapeDtypeStruct((C, D), _f32),
    )(w0.reshape(N, 1), w1.reshape(N, 1), *tw, b_wo.reshape(1, D))

    w0_leaf, w1_leaf, idrows, src = _gather_call(
        w0, w1, leaf_nodes, identifiers_encodings,
        ii(id_leaf_identifier_idx))

    nid = P_ID // TILE
    r_of = lambda i: jnp.where(i < nid, 0,
                               jnp.where(i < nid + P_PT // TILE, 1, 2))
    combined = pl.pallas_call(
        _leaf_body,
        grid=(P // TILE,),
        in_specs=[
            pl.BlockSpec(memory_space=pl.ANY),
            pl.BlockSpec((TILE, 1), lambda i: (i, 0)),
            pl.BlockSpec((TILE, 1), lambda i: (i, 0)),
            pl.BlockSpec((TILE, D), lambda i: (jnp.minimum(i, nid - 1), 0)),
            pl.BlockSpec((TILE, 1),
                         lambda i: (jnp.clip(i - nid, 0, nid - 1), 0)),
            full((D, D)),
            pl.BlockSpec((1, 200, D), lambda i: (r_of(i), 0, 0)),
            pl.BlockSpec((1, 32, D), lambda i: (r_of(i), 0, 0)),
            pl.BlockSpec((1, 64, D), lambda i: (r_of(i), 0, 0)),
            pl.BlockSpec((1, 32, D), lambda i: (r_of(i), 0, 0)),
            pl.BlockSpec((1, 64, D), lambda i: (r_of(i), 0, 0)),
            pl.BlockSpec((1, 16, D), lambda i: (r_of(i), 0, 0)),
            pl.BlockSpec((1, 1, D), lambda i: (r_of(i), 0, 0)),
        ],
        out_specs=pl.BlockSpec((TILE, D), lambda i: (i, 0)),
        out_shape=jax.ShapeDtypeStruct((C, D), _f32),
        input_output_aliases={0: 0},
    )(combined0, w0_leaf.reshape(P, 1), w1_leaf.reshape(P, 1), idrows,
      featpm, w_ida, *stk, ft_stk, b_stk)

    return _emit_call(src, combined)


# speedup vs baseline: 15.8004x; 1.0245x over previous
import functools

import jax
import jax.numpy as jnp
from jax import lax
from jax.experimental import pallas as pl
from jax.experimental.pallas import tpu as pltpu
from jax.experimental.pallas import tpu_sc as plsc

N = 262144
P_ID = 65536
P_PT = 32768
P_MOD = 32768
P = P_ID + P_PT + P_MOD
C = P + N
D = 128
PAD = 0

TILE = 1024
NC = 2
NS = 16
NW = NC * NS

_f32 = jnp.float32
_i32 = jnp.int32
_u8 = jnp.uint8


def _oh(idx_col, v, dtype=_f32):
    rows = idx_col.shape[0]
    io = lax.broadcasted_iota(_i32, (rows, v), 1)
    return (idx_col == io).astype(dtype)


def _unpack(w0, w1):
    typ = w0 & 255
    maj = (w0 >> 8) & 31
    mnr = (w0 >> 13) & 63
    nrc = (w0 >> 19) & 31
    ltr = w1 & 63
    rtl = (w1 >> 6) & 63
    return typ, maj, mnr, nrc, ltr, rtl


def _base_body(w0_ref, w1_ref, t1_ref, t2_ref, t3_ref, t4_ref, t5_ref, b_ref,
               out_ref):
    typ, maj, mnr, nrc, ltr, rtl = _unpack(w0_ref[...], w1_ref[...])
    first = jnp.where(
        mnr == PAD,
        jnp.dot(_oh(typ, 200), t1_ref[...], preferred_element_type=_f32),
        jnp.dot(_oh(maj, 32), t2_ref[...], preferred_element_type=_f32)
        + jnp.dot(_oh(mnr, 64), t3_ref[...], preferred_element_type=_f32),
    )
    acc = (first
           + jnp.dot(_oh(nrc, 32), t4_ref[...], preferred_element_type=_f32)
           + jnp.dot(_oh(ltr, 64) + _oh(rtl, 64), t5_ref[...],
                     preferred_element_type=_f32)
           + b_ref[...])
    out_ref[...] = jnp.maximum(acc, 0.0)


def _leaf_body(comb_in_ref, w0_ref, w1_ref, idenc_ref, featpm_ref, wida_ref,
               t1_ref, t2_ref, t3_ref, t4_ref, t5_ref, ft_ref, b_ref,
               out_ref):
    del comb_in_ref
    i = pl.program_id(0)
    typ, maj, mnr, nrc, ltr, rtl = _unpack(w0_ref[...], w1_ref[...])
    first = jnp.where(
        mnr == PAD,
        jnp.dot(_oh(typ, 200), t1_ref[0], preferred_element_type=_f32),
        jnp.dot(_oh(maj, 32), t2_ref[0], preferred_element_type=_f32)
        + jnp.dot(_oh(mnr, 64), t3_ref[0], preferred_element_type=_f32),
    )
    acc = (first
           + jnp.dot(_oh(nrc, 32), t4_ref[0], preferred_element_type=_f32)
           + jnp.dot(_oh(ltr, 64) + _oh(rtl, 64), t5_ref[0],
                     preferred_element_type=_f32)
           + b_ref[0])

    @pl.when(i < P_ID // TILE)
    def _id_region():
        feat = jnp.dot(idenc_ref[...], wida_ref[...],
                       preferred_element_type=_f32)
        out_ref[...] = jnp.maximum(acc + feat, 0.0)

    @pl.when(i >= P_ID // TILE)
    def _pm_region():
        feat = jnp.dot(_oh(featpm_ref[...], 16), ft_ref[0],
                       preferred_element_type=_f32)
        out_ref[...] = jnp.maximum(acc + feat, 0.0)


def _sc_gather(w0_hbm, w1_hbm, lnodes_hbm, ident_hbm, ididx_hbm,
               w0_out, w1_out, idrows_out,
               idxa_v, idxi_v, b0, b1, rbuf, sem):
    wid = lax.axis_index("s") * NC + lax.axis_index("c")
    a0 = wid * (P // NW)
    pltpu.sync_copy(lnodes_hbm.at[pl.ds(a0, 4096)], idxa_v)

    def _ga(k, _):
        sl = pl.ds(k * 128, 128)
        pltpu.async_copy(w0_hbm.at[idxa_v.at[sl]], b0.at[sl], sem).wait()
        pltpu.async_copy(w1_hbm.at[idxa_v.at[sl]], b1.at[sl], sem).wait()
        return 0
    lax.fori_loop(0, 32, _ga, 0)
    pltpu.sync_copy(b0, w0_out.at[pl.ds(a0, 4096)])
    pltpu.sync_copy(b1, w1_out.at[pl.ds(a0, 4096)])
    i0 = wid * (P_ID // NW)
    pltpu.sync_copy(ididx_hbm.at[pl.ds(i0, 2048)], idxi_v)
    for oc in range(4):
        def _gi(k, _, oc=oc):
            pltpu.async_copy(
                ident_hbm.at[idxi_v.at[pl.ds(oc * 512 + k * 128, 128)]],
                rbuf.at[pl.ds(k * 128, 128)], sem).wait()
            return 0
        lax.fori_loop(0, 4, _gi, 0)
        pltpu.sync_copy(rbuf, idrows_out.at[pl.ds(i0 + oc * 512, 512)])


def _sc_winner(lnodes_hbm, src_out, aux_v, sbuf, srcb, tmp16):
    wid = lax.axis_index("s") * NC + lax.axis_index("c")
    R = N // NW
    lo = wid * R
    zeros16 = jnp.zeros((16,), _i32)

    def _z(j, _):
        aux_v[pl.ds(j * 16, 16)] = zeros16
        return 0
    lax.fori_loop(0, R // 16, _z, 0)
    iota16 = lax.iota(_i32, 16)
    nxtidx = jnp.minimum(iota16 + 1, 15)

    def _one(cb, j):
        v = sbuf[pl.ds(j * 16, 16)]
        m = (v >= lo) & (v < lo + R)

        @pl.when(jnp.any(m))
        def _hit():
            key = cb * 4096 + j * 16 + 1 + iota16
            c = jnp.where(m, ((v - lo) << 18) | key, -1)
            sc, _unused = plsc.sort_key_val(c, c)
            tmp16[...] = sc
            nxt = plsc.load_gather(tmp16, [nxtidx])
            islast = ((sc >> 18) != (nxt >> 18)) | (iota16 == 15)
            upd = islast & (sc >= 0)
            addr = jnp.maximum(sc >> 18, 0)
            plsc.store_scatter(aux_v, [addr], sc & 0x3FFFF, mask=upd)

    def _outer(cb, _):
        pltpu.sync_copy(lnodes_hbm.at[pl.ds(cb * 4096, 4096)], sbuf)

        def _inner(jj, _):
            _one(cb, 2 * jj)
            _one(cb, 2 * jj + 1)
            return 0
        lax.fori_loop(0, 128, _inner, 0)
        return 0
    lax.fori_loop(0, P // 4096, _outer, 0)

    def _src(q, _):
        def _sv(j, _):
            k = aux_v[pl.ds(q * 2048 + j * 16, 16)]
            node = lo + q * 2048 + j * 16 + iota16
            srcb[pl.ds(j * 16, 16)] = jnp.where(k > 0, k - 1, node + P)
            return 0
        lax.fori_loop(0, 128, _sv, 0)
        pltpu.sync_copy(srcb, src_out.at[pl.ds(lo + q * 2048, 2048)])
        return 0
    lax.fori_loop(0, R // 2048, _src, 0)


def _sc_emit(src_hbm, comb_hbm, out_hbm, srcv, rbuf, sem):
    wid = lax.axis_index("s") * NC + lax.axis_index("c")
    n0 = wid * (N // NW)

    def _chunk(ch, _):
        base = n0 + ch * 512
        pltpu.sync_copy(src_hbm.at[pl.ds(base, 512)], srcv)

        def _g(k, _):
            pltpu.async_copy(
                comb_hbm.at[srcv.at[pl.ds(k * 128, 128)]],
                rbuf.at[pl.ds(k * 128, 128)], sem).wait()
            return 0
        lax.fori_loop(0, 4, _g, 0)
        pltpu.sync_copy(rbuf, out_hbm.at[pl.ds(base, 512)])
        return 0
    lax.fori_loop(0, N // NW // 512, _chunk, 0)


def _sc_mesh():
    return plsc.VectorSubcoreMesh(core_axis_name="c", subcore_axis_name="s",
                                  num_cores=NC, num_subcores=NS)


def _gather_call(*args):
    return pl.kernel(
        _sc_gather,
        out_type=(jax.ShapeDtypeStruct((P,), _i32),
                  jax.ShapeDtypeStruct((P,), _i32),
                  jax.ShapeDtypeStruct((P_ID, D), _f32)),
        scratch_types=[
            pltpu.VMEM((4096,), _i32),
            pltpu.VMEM((2048,), _i32),
            pltpu.VMEM((4096,), _i32),
            pltpu.VMEM((4096,), _i32),
            pltpu.VMEM((512, D), _f32),
            pltpu.SemaphoreType.DMA,
        ],
        mesh=_sc_mesh(),
        compiler_params=pltpu.CompilerParams(needs_layout_passes=False),
    )(*args)


def _winner_call(*args):
    return pl.kernel(
        _sc_winner,
        out_type=jax.ShapeDtypeStruct((N,), _i32),
        scratch_types=[
            pltpu.VMEM((N // NW,), _i32),
            pltpu.VMEM((4096,), _i32),
            pltpu.VMEM((2048,), _i32),
            pltpu.VMEM((16,), _i32),
        ],
        mesh=_sc_mesh(),
        compiler_params=pltpu.CompilerParams(needs_layout_passes=False),
    )(*args)


def _emit_call(*args):
    return pl.kernel(
        _sc_emit,
        out_type=jax.ShapeDtypeStruct((N, D), _f32),
        scratch_types=[
            pltpu.VMEM((512,), _i32),
            pltpu.VMEM((512, D), _f32),
            pltpu.SemaphoreType.DMA,
        ],
        mesh=_sc_mesh(),
        compiler_params=pltpu.CompilerParams(needs_layout_passes=False),
    )(*args)


def _fold(table, w_part):
    return jnp.dot(table, w_part, preferred_element_type=_f32)


def _fold_stage(type_emb, major_emb, minor_emb, nrc_emb, pos_emb, w):
    a, b, c = w[0:128], w[128:160], w[160:192]
    t1 = _fold(type_emb, a)
    t2 = jnp.pad(_fold(major_emb, a[0:85]), ((0, 2), (0, 0)))
    t3 = jnp.pad(_fold(minor_emb, a[85:128]), ((0, 4), (0, 0)))
    t4 = _fold(nrc_emb, b)
    t5 = _fold(pos_emb, c)
    return t1, t2, t3, t4, t5


def kernel(ast_node_types, ast_node_major_types, ast_node_minor_types,
           ast_node_nr_children, ast_node_child_ltr_position,
           ast_node_child_rtl_position, id_leaf_node_indices,
           id_leaf_identifier_idx, prim_leaf_node_indices,
           prim_leaf_primitive_type, mod_leaf_node_indices, mod_leaf_modifier,
           identifiers_encodings, type_emb, major_emb, minor_emb, nrc_emb,
           pos_emb, prim_emb, mod_emb, W_id, b_id, W_prim, b_prim, W_mod,
           b_mod, W_wo, b_wo):
    ii = lambda x: x.astype(_i32)
    w0 = (ii(ast_node_types) | (ii(ast_node_major_types) << 8)
          | (ii(ast_node_minor_types) << 13) | (ii(ast_node_nr_children) << 19))
    w1 = ii(ast_node_child_ltr_position) | (ii(ast_node_child_rtl_position) << 6)
    leaf_nodes = jnp.concatenate([ii(id_leaf_node_indices),
                                  ii(prim_leaf_node_indices),
                                  ii(mod_leaf_node_indices)])
    featpm = jnp.concatenate([ii(prim_leaf_primitive_type),
                              ii(mod_leaf_modifier)]).reshape(-1, 1)

    tw = _fold_stage(type_emb, major_emb, minor_emb, nrc_emb, pos_emb, W_wo)
    tid = _fold_stage(type_emb, major_emb, minor_emb, nrc_emb, pos_emb,
                      W_id[D:])
    tpr = _fold_stage(type_emb, major_emb, minor_emb, nrc_emb, pos_emb,
                      W_prim[64:])
    tmo = _fold_stage(type_emb, major_emb, minor_emb, nrc_emb, pos_emb,
                      W_mod[64:])
    stk = [jnp.stack([tid[j], tpr[j], tmo[j]]) for j in range(5)]
    ft_stk = jnp.stack([jnp.zeros((16, D), _f32),
                        _fold(prim_emb, W_prim[0:64]),
                        _fold(mod_emb, W_mod[0:64])])
    b_stk = jnp.stack([b_id.reshape(1, D), b_prim.reshape(1, D),
                       b_mod.reshape(1, D)])
    w_ida = W_id[0:D]

    full = lambda s: pl.BlockSpec(s, lambda i: (0,) * len(s))
    combined0 = pl.pallas_call(
        _base_body,
        grid=(N // TILE,),
        in_specs=[
            pl.BlockSpec((TILE, 1), lambda i: (i, 0)),
            pl.BlockSpec((TILE, 1), lambda i: (i, 0)),
            full((200, D)), full((32, D)), full((64, D)), full((32, D)),
            full((64, D)), full((1, D)),
        ],
        out_specs=pl.BlockSpec((TILE, D), lambda i: (i + P // TILE, 0)),
        out_shape=jax.ShapeDtypeStruct((C, D), _f32),
    )(w0.reshape(N, 1), w1.reshape(N, 1), *tw, b_wo.reshape(1, D))

    w0_leaf, w1_leaf, idrows = _gather_call(
        w0, w1, leaf_nodes, identifiers_encodings,
        ii(id_leaf_identifier_idx))
    src = _winner_call(leaf_nodes)

    nid = P_ID // TILE
    r_of = lambda i: jnp.where(i < nid, 0,
                               jnp.where(i < nid + P_PT // TILE, 1, 2))
    combined = pl.pallas_call(
        _leaf_body,
        grid=(P // TILE,),
        in_specs=[
            pl.BlockSpec(memory_space=pl.ANY),
            pl.BlockSpec((TILE, 1), lambda i: (i, 0)),
            pl.BlockSpec((TILE, 1), lambda i: (i, 0)),
            pl.BlockSpec((TILE, D), lambda i: (jnp.minimum(i, nid - 1), 0)),
            pl.BlockSpec((TILE, 1),
                         lambda i: (jnp.clip(i - nid, 0, nid - 1), 0)),
            full((D, D)),
            pl.BlockSpec((1, 200, D), lambda i: (r_of(i), 0, 0)),
            pl.BlockSpec((1, 32, D), lambda i: (r_of(i), 0, 0)),
            pl.BlockSpec((1, 64, D), lambda i: (r_of(i), 0, 0)),
            pl.BlockSpec((1, 32, D), lambda i: (r_of(i), 0, 0)),
            pl.BlockSpec((1, 64, D), lambda i: (r_of(i), 0, 0)),
            pl.BlockSpec((1, 16, D), lambda i: (r_of(i), 0, 0)),
            pl.BlockSpec((1, 1, D), lambda i: (r_of(i), 0, 0)),
        ],
        out_specs=pl.BlockSpec((TILE, D), lambda i: (i, 0)),
        out_shape=jax.ShapeDtypeStruct((C, D), _f32),
        input_output_aliases={0: 0},
    )(combined0, w0_leaf.reshape(P, 1), w1_leaf.reshape(P, 1), idrows,
      featpm, w_ida, *stk, ft_stk, b_stk)

    return _emit_call(src, combined)


# speedup vs baseline: 16.0216x; 1.0140x over previous
import functools

import jax
import jax.numpy as jnp
from jax import lax
from jax.experimental import pallas as pl
from jax.experimental.pallas import tpu as pltpu
from jax.experimental.pallas import tpu_sc as plsc

N = 262144
P_ID = 65536
P_PT = 32768
P_MOD = 32768
P = P_ID + P_PT + P_MOD
C = P + N
D = 128
PAD = 0

TILE = 1024
NC = 2
NS = 16
NW = NC * NS

_f32 = jnp.float32
_i32 = jnp.int32
_u8 = jnp.uint8


def _oh(idx_col, v, dtype=_f32):
    rows = idx_col.shape[0]
    io = lax.broadcasted_iota(_i32, (rows, v), 1)
    return (idx_col == io).astype(dtype)


def _unpack(w0, w1):
    typ = w0 & 255
    maj = (w0 >> 8) & 31
    mnr = (w0 >> 13) & 63
    nrc = (w0 >> 19) & 31
    ltr = w1 & 63
    rtl = (w1 >> 6) & 63
    return typ, maj, mnr, nrc, ltr, rtl


def _base_body(w0_ref, w1_ref, t1_ref, t2_ref, t3_ref, t4_ref, t5_ref, b_ref,
               out_ref):
    typ, maj, mnr, nrc, ltr, rtl = _unpack(w0_ref[...], w1_ref[...])
    first = jnp.where(
        mnr == PAD,
        jnp.dot(_oh(typ, 200), t1_ref[...], preferred_element_type=_f32),
        jnp.dot(_oh(maj, 32), t2_ref[...], preferred_element_type=_f32)
        + jnp.dot(_oh(mnr, 64), t3_ref[...], preferred_element_type=_f32),
    )
    acc = (first
           + jnp.dot(_oh(nrc, 32), t4_ref[...], preferred_element_type=_f32)
           + jnp.dot(_oh(ltr, 64) + _oh(rtl, 64), t5_ref[...],
                     preferred_element_type=_f32)
           + b_ref[...])
    out_ref[...] = jnp.maximum(acc, 0.0)


def _leaf_body(comb_in_ref, w0_ref, w1_ref, idenc_ref, featpm_ref, wida_ref,
               t1_ref, t2_ref, t3_ref, t4_ref, t5_ref, ft_ref, b_ref,
               out_ref):
    del comb_in_ref
    i = pl.program_id(0)
    typ, maj, mnr, nrc, ltr, rtl = _unpack(w0_ref[...], w1_ref[...])
    first = jnp.where(
        mnr == PAD,
        jnp.dot(_oh(typ, 200), t1_ref[0], preferred_element_type=_f32),
        jnp.dot(_oh(maj, 32), t2_ref[0], preferred_element_type=_f32)
        + jnp.dot(_oh(mnr, 64), t3_ref[0], preferred_element_type=_f32),
    )
    acc = (first
           + jnp.dot(_oh(nrc, 32), t4_ref[0], preferred_element_type=_f32)
           + jnp.dot(_oh(ltr, 64) + _oh(rtl, 64), t5_ref[0],
                     preferred_element_type=_f32)
           + b_ref[0])

    @pl.when(i < P_ID // TILE)
    def _id_region():
        feat = jnp.dot(idenc_ref[...], wida_ref[...],
                       preferred_element_type=_f32)
        out_ref[...] = jnp.maximum(acc + feat, 0.0)

    @pl.when(i >= P_ID // TILE)
    def _pm_region():
        feat = jnp.dot(_oh(featpm_ref[...], 16), ft_ref[0],
                       preferred_element_type=_f32)
        out_ref[...] = jnp.maximum(acc + feat, 0.0)


def _sc_fused(w0_hbm, w1_hbm, lnodes_hbm, ident_hbm, ididx_hbm,
              w0_out, w1_out, idrows_out, src_out,
              idxa_v, idxi_v, b0, b1, rbuf, aux_v, sbuf, srcb, tmp16, sem):
    wid = lax.axis_index("s") * NC + lax.axis_index("c")
    iota16 = lax.iota(_i32, 16)
    nxtidx = jnp.minimum(iota16 + 1, 15)

    @pl.when(wid < 16)
    def _gather_half():
        g = wid
        a0 = g * (P // 16)
        pltpu.sync_copy(lnodes_hbm.at[pl.ds(a0, 8192)], idxa_v)

        def _ga(k, _):
            sl = pl.ds(k * 128, 128)
            pltpu.async_copy(w0_hbm.at[idxa_v.at[sl]], b0.at[sl], sem).wait()
            pltpu.async_copy(w1_hbm.at[idxa_v.at[sl]], b1.at[sl], sem).wait()
            return 0
        lax.fori_loop(0, 64, _ga, 0)
        pltpu.sync_copy(b0, w0_out.at[pl.ds(a0, 8192)])
        pltpu.sync_copy(b1, w1_out.at[pl.ds(a0, 8192)])
        i0 = g * (P_ID // 16)
        pltpu.sync_copy(ididx_hbm.at[pl.ds(i0, 4096)], idxi_v)
        for oc in range(8):
            def _gi(k, _, oc=oc):
                pltpu.async_copy(
                    ident_hbm.at[idxi_v.at[pl.ds(oc * 512 + k * 128, 128)]],
                    rbuf.at[pl.ds(k * 128, 128)], sem).wait()
                return 0
            lax.fori_loop(0, 4, _gi, 0)
            pltpu.sync_copy(rbuf, idrows_out.at[pl.ds(i0 + oc * 512, 512)])

    @pl.when(wid >= 16)
    def _winner_half():
        R = N // 16
        lo = (wid - 16) * R
        neg16 = jnp.full((16,), -1, _i32)

        def _z(j, _):
            aux_v[pl.ds(j * 16, 16)] = neg16
            return 0
        lax.fori_loop(0, R // 16, _z, 0)

        def _one(cb, j):
            v = sbuf[pl.ds(j * 16, 16)]
            m = (v >= lo) & (v < lo + R)

            @pl.when(jnp.any(m))
            def _hit():
                pos = cb * 4096 + j * 16 + iota16
                c = jnp.where(m, ((v - lo) << 17) | pos, -1)
                sc, _unused = plsc.sort_key_val(c, c)
                tmp16[...] = sc
                nxt = plsc.load_gather(tmp16, [nxtidx])
                islast = ((sc >> 17) != (nxt >> 17)) | (iota16 == 15)
                upd = islast & (sc >= 0)
                addr = jnp.maximum(sc >> 17, 0)
                plsc.store_scatter(aux_v, [addr], sc & 0x1FFFF, mask=upd)

        def _outer(cb, _):
            pltpu.sync_copy(lnodes_hbm.at[pl.ds(cb * 4096, 4096)], sbuf)

            def _inner(jj, _):
                _one(cb, 2 * jj)
                _one(cb, 2 * jj + 1)
                return 0
            lax.fori_loop(0, 128, _inner, 0)
            return 0
        lax.fori_loop(0, P // 4096, _outer, 0)

        def _src(q, _):
            def _sv(j, _):
                k = aux_v[pl.ds(q * 2048 + j * 16, 16)]
                node = lo + q * 2048 + j * 16 + iota16
                srcb[pl.ds(j * 16, 16)] = jnp.where(k >= 0, k, node + P)
                return 0
            lax.fori_loop(0, 128, _sv, 0)
            pltpu.sync_copy(srcb, src_out.at[pl.ds(lo + q * 2048, 2048)])
            return 0
        lax.fori_loop(0, R // 2048, _src, 0)


def _sc_emit(src_hbm, comb_hbm, out_hbm, srcv, rbuf, sem):
    wid = lax.axis_index("s") * NC + lax.axis_index("c")
    n0 = wid * (N // NW)

    def _chunk(ch, _):
        base = n0 + ch * 512
        pltpu.sync_copy(src_hbm.at[pl.ds(base, 512)], srcv)

        def _g(k, _):
            pltpu.async_copy(
                comb_hbm.at[srcv.at[pl.ds(k * 128, 128)]],
                rbuf.at[pl.ds(k * 128, 128)], sem).wait()
            return 0
        lax.fori_loop(0, 4, _g, 0)
        pltpu.sync_copy(rbuf, out_hbm.at[pl.ds(base, 512)])
        return 0
    lax.fori_loop(0, N // NW // 512, _chunk, 0)


def _sc_mesh():
    return plsc.VectorSubcoreMesh(core_axis_name="c", subcore_axis_name="s",
                                  num_cores=NC, num_subcores=NS)


def _fused_call(*args):
    return pl.kernel(
        _sc_fused,
        out_type=(jax.ShapeDtypeStruct((P,), _i32),
                  jax.ShapeDtypeStruct((P,), _i32),
                  jax.ShapeDtypeStruct((P_ID, D), _f32),
                  jax.ShapeDtypeStruct((N,), _i32)),
        scratch_types=[
            pltpu.VMEM((8192,), _i32),
            pltpu.VMEM((4096,), _i32),
            pltpu.VMEM((8192,), _i32),
            pltpu.VMEM((8192,), _i32),
            pltpu.VMEM((512, D), _f32),
            pltpu.VMEM((N // 16,), _i32),
            pltpu.VMEM((4096,), _i32),
            pltpu.VMEM((2048,), _i32),
            pltpu.VMEM((16,), _i32),
            pltpu.SemaphoreType.DMA,
        ],
        mesh=_sc_mesh(),
        compiler_params=pltpu.CompilerParams(needs_layout_passes=False),
    )(*args)


def _emit_call(*args):
    return pl.kernel(
        _sc_emit,
        out_type=jax.ShapeDtypeStruct((N, D), _f32),
        scratch_types=[
            pltpu.VMEM((512,), _i32),
            pltpu.VMEM((512, D), _f32),
            pltpu.SemaphoreType.DMA,
        ],
        mesh=_sc_mesh(),
        compiler_params=pltpu.CompilerParams(needs_layout_passes=False),
    )(*args)


def _fold(table, w_part):
    return jnp.dot(table, w_part, preferred_element_type=_f32)


def _fold_stage(type_emb, major_emb, minor_emb, nrc_emb, pos_emb, w):
    a, b, c = w[0:128], w[128:160], w[160:192]
    t1 = _fold(type_emb, a)
    t2 = jnp.pad(_fold(major_emb, a[0:85]), ((0, 2), (0, 0)))
    t3 = jnp.pad(_fold(minor_emb, a[85:128]), ((0, 4), (0, 0)))
    t4 = _fold(nrc_emb, b)
    t5 = _fold(pos_emb, c)
    return t1, t2, t3, t4, t5


def kernel(ast_node_types, ast_node_major_types, ast_node_minor_types,
           ast_node_nr_children, ast_node_child_ltr_position,
           ast_node_child_rtl_position, id_leaf_node_indices,
           id_leaf_identifier_idx, prim_leaf_node_indices,
           prim_leaf_primitive_type, mod_leaf_node_indices, mod_leaf_modifier,
           identifiers_encodings, type_emb, major_emb, minor_emb, nrc_emb,
           pos_emb, prim_emb, mod_emb, W_id, b_id, W_prim, b_prim, W_mod,
           b_mod, W_wo, b_wo):
    ii = lambda x: x.astype(_i32)
    w0 = (ii(ast_node_types) | (ii(ast_node_major_types) << 8)
          | (ii(ast_node_minor_types) << 13) | (ii(ast_node_nr_children) << 19))
    w1 = ii(ast_node_child_ltr_position) | (ii(ast_node_child_rtl_position) << 6)
    leaf_nodes = jnp.concatenate([ii(id_leaf_node_indices),
                                  ii(prim_leaf_node_indices),
                                  ii(mod_leaf_node_indices)])
    featpm = jnp.concatenate([ii(prim_leaf_primitive_type),
                              ii(mod_leaf_modifier)]).reshape(-1, 1)

    tw = _fold_stage(type_emb, major_emb, minor_emb, nrc_emb, pos_emb, W_wo)
    tid = _fold_stage(type_emb, major_emb, minor_emb, nrc_emb, pos_emb,
                      W_id[D:])
    tpr = _fold_stage(type_emb, major_emb, minor_emb, nrc_emb, pos_emb,
                      W_prim[64:])
    tmo = _fold_stage(type_emb, major_emb, minor_emb, nrc_emb, pos_emb,
                      W_mod[64:])
    stk = [jnp.stack([tid[j], tpr[j], tmo[j]]) for j in range(5)]
    ft_stk = jnp.stack([jnp.zeros((16, D), _f32),
                        _fold(prim_emb, W_prim[0:64]),
                        _fold(mod_emb, W_mod[0:64])])
    b_stk = jnp.stack([b_id.reshape(1, D), b_prim.reshape(1, D),
                       b_mod.reshape(1, D)])
    w_ida = W_id[0:D]

    full = lambda s: pl.BlockSpec(s, lambda i: (0,) * len(s))
    combined0 = pl.pallas_call(
        _base_body,
        grid=(N // TILE,),
        in_specs=[
            pl.BlockSpec((TILE, 1), lambda i: (i, 0)),
            pl.BlockSpec((TILE, 1), lambda i: (i, 0)),
            full((200, D)), full((32, D)), full((64, D)), full((32, D)),
            full((64, D)), full((1, D)),
        ],
        out_specs=pl.BlockSpec((TILE, D), lambda i: (i + P // TILE, 0)),
        out_shape=jax.ShapeDtypeStruct((C, D), _f32),
    )(w0.reshape(N, 1), w1.reshape(N, 1), *tw, b_wo.reshape(1, D))

    w0_leaf, w1_leaf, idrows, src = _fused_call(
        w0, w1, leaf_nodes, identifiers_encodings,
        ii(id_leaf_identifier_idx))

    nid = P_ID // TILE
    r_of = lambda i: jnp.where(i < nid, 0,
                               jnp.where(i < nid + P_PT // TILE, 1, 2))
    combined = pl.pallas_call(
        _leaf_body,
        grid=(P // TILE,),
        in_specs=[
            pl.BlockSpec(memory_space=pl.ANY),
            pl.BlockSpec((TILE, 1), lambda i: (i, 0)),
            pl.BlockSpec((TILE, 1), lambda i: (i, 0)),
            pl.BlockSpec((TILE, D), lambda i: (jnp.minimum(i, nid - 1), 0)),
            pl.BlockSpec((TILE, 1),
                         lambda i: (jnp.clip(i - nid, 0, nid - 1), 0)),
            full((D, D)),
            pl.BlockSpec((1, 200, D), lambda i: (r_of(i), 0, 0)),
            pl.BlockSpec((1, 32, D), lambda i: (r_of(i), 0, 0)),
            pl.BlockSpec((1, 64, D), lambda i: (r_of(i), 0, 0)),
            pl.BlockSpec((1, 32, D), lambda i: (r_of(i), 0, 0)),
            pl.BlockSpec((1, 64, D), lambda i: (r_of(i), 0, 0)),
            pl.BlockSpec((1, 16, D), lambda i: (r_of(i), 0, 0)),
            pl.BlockSpec((1, 1, D), lambda i: (r_of(i), 0, 0)),
        ],
        out_specs=pl.BlockSpec((TILE, D), lambda i: (i, 0)),
        out_shape=jax.ShapeDtypeStruct((C, D), _f32),
        input_output_aliases={0: 0},
    )(combined0, w0_leaf.reshape(P, 1), w1_leaf.reshape(P, 1), idrows,
      featpm, w_ida, *stk, ft_stk, b_stk)

    return _emit_call(src, combined)


# speedup vs baseline: 16.0357x; 1.0009x over previous
import jax
import jax.numpy as jnp
from jax import lax
from jax.experimental import pallas as pl
from jax.experimental.pallas import tpu as pltpu
from jax.experimental.pallas import tpu_sc as plsc

N = 262144
P_ID = 65536
P_PT = 32768
P_MOD = 32768
P = P_ID + P_PT + P_MOD
C = P + N
D = 128
PAD = 0

TILE = 1024
NC = 2
NS = 16
NW = NC * NS

_f32 = jnp.float32
_i32 = jnp.int32


def _oh(idx_col, v, dtype=_f32):
    rows = idx_col.shape[0]
    io = lax.broadcasted_iota(_i32, (rows, v), 1)
    return (idx_col == io).astype(dtype)


def _unpack(w0, w1):
    typ = w0 & 255
    maj = (w0 >> 8) & 31
    mnr = (w0 >> 13) & 63
    nrc = (w0 >> 19) & 31
    ltr = w1 & 63
    rtl = (w1 >> 6) & 63
    return typ, maj, mnr, nrc, ltr, rtl


def _base_body(w0_ref, w1_ref, t1_ref, t2_ref, t3_ref, t4_ref, t5_ref, b_ref,
               out_ref):
    typ, maj, mnr, nrc, ltr, rtl = _unpack(w0_ref[...], w1_ref[...])
    first = jnp.where(
        mnr == PAD,
        jnp.dot(_oh(typ, 200), t1_ref[...], preferred_element_type=_f32),
        jnp.dot(_oh(maj, 32), t2_ref[...], preferred_element_type=_f32)
        + jnp.dot(_oh(mnr, 64), t3_ref[...], preferred_element_type=_f32),
    )
    acc = (first
           + jnp.dot(_oh(nrc, 32), t4_ref[...], preferred_element_type=_f32)
           + jnp.dot(_oh(ltr, 64) + _oh(rtl, 64), t5_ref[...],
                     preferred_element_type=_f32)
           + b_ref[...])
    out_ref[...] = jnp.maximum(acc, 0.0)


def _leaf_body(comb_in_ref, w0_ref, w1_ref, idenc_ref, featpm_ref, wida_ref,
               t1_ref, t2_ref, t3_ref, t4_ref, t5_ref, ft_ref, b_ref,
               out_ref):
    del comb_in_ref
    i = pl.program_id(0)
    typ, maj, mnr, nrc, ltr, rtl = _unpack(w0_ref[...], w1_ref[...])
    first = jnp.where(
        mnr == PAD,
        jnp.dot(_oh(typ, 200), t1_ref[0], preferred_element_type=_f32),
        jnp.dot(_oh(maj, 32), t2_ref[0], preferred_element_type=_f32)
        + jnp.dot(_oh(mnr, 64), t3_ref[0], preferred_element_type=_f32),
    )
    acc = (first
           + jnp.dot(_oh(nrc, 32), t4_ref[0], preferred_element_type=_f32)
           + jnp.dot(_oh(ltr, 64) + _oh(rtl, 64), t5_ref[0],
                     preferred_element_type=_f32)
           + b_ref[0])

    @pl.when(i < P_ID // TILE)
    def _id_region():
        feat = jnp.dot(idenc_ref[...], wida_ref[...],
                       preferred_element_type=_f32)
        out_ref[...] = jnp.maximum(acc + feat, 0.0)

    @pl.when(i >= P_ID // TILE)
    def _pm_region():
        feat = jnp.dot(_oh(featpm_ref[...], 16), ft_ref[0],
                       preferred_element_type=_f32)
        out_ref[...] = jnp.maximum(acc + feat, 0.0)


def _sc_fused(w0_hbm, w1_hbm, lnodes_hbm, ident_hbm, ididx_hbm,
              w0_out, w1_out, idrows_out, src_out,
              idxa_v, idxi_v, b0, b1, rbuf, aux_v, sbuf, srcb, tmp16, sem):
    wid = lax.axis_index("s") * NC + lax.axis_index("c")
    iota16 = lax.iota(_i32, 16)
    nxtidx = jnp.minimum(iota16 + 1, 15)

    @pl.when(wid < 16)
    def _gather_half():
        g = wid
        a0 = g * (P // 16)
        pltpu.sync_copy(lnodes_hbm.at[pl.ds(a0, 8192)], idxa_v)

        def _ga(k, _):
            sl = pl.ds(k * 128, 128)
            pltpu.async_copy(w0_hbm.at[idxa_v.at[sl]], b0.at[sl], sem).wait()
            pltpu.async_copy(w1_hbm.at[idxa_v.at[sl]], b1.at[sl], sem).wait()
            return 0
        lax.fori_loop(0, 64, _ga, 0)
        pltpu.sync_copy(b0, w0_out.at[pl.ds(a0, 8192)])
        pltpu.sync_copy(b1, w1_out.at[pl.ds(a0, 8192)])
        i0 = g * (P_ID // 16)
        pltpu.sync_copy(ididx_hbm.at[pl.ds(i0, 4096)], idxi_v)
        for oc in range(8):
            def _gi(k, _, oc=oc):
                pltpu.async_copy(
                    ident_hbm.at[idxi_v.at[pl.ds(oc * 512 + k * 128, 128)]],
                    rbuf.at[pl.ds(k * 128, 128)], sem).wait()
                return 0
            lax.fori_loop(0, 4, _gi, 0)
            pltpu.sync_copy(rbuf, idrows_out.at[pl.ds(i0 + oc * 512, 512)])

    @pl.when(wid >= 16)
    def _winner_half():
        R = N // 16
        lo = (wid - 16) * R
        neg16 = jnp.full((16,), -1, _i32)

        def _z(j, _):
            aux_v[pl.ds(j * 16, 16)] = neg16
            return 0
        lax.fori_loop(0, R // 16, _z, 0)

        def _one(cb, j):
            v = sbuf[pl.ds(j * 16, 16)]
            m = (v >= lo) & (v < lo + R)

            @pl.when(jnp.any(m))
            def _hit():
                pos = cb * 4096 + j * 16 + iota16
                c = jnp.where(m, ((v - lo) << 17) | pos, -1)
                sc, _unused = plsc.sort_key_val(c, c)
                tmp16[...] = sc
                nxt = plsc.load_gather(tmp16, [nxtidx])
                islast = ((sc >> 17) != (nxt >> 17)) | (iota16 == 15)
                upd = islast & (sc >= 0)
                addr = jnp.maximum(sc >> 17, 0)
                plsc.store_scatter(aux_v, [addr], sc & 0x1FFFF, mask=upd)

        def _outer(cb, _):
            pltpu.sync_copy(lnodes_hbm.at[pl.ds(cb * 4096, 4096)], sbuf)

            def _inner(jj, _):
                _one(cb, 2 * jj)
                _one(cb, 2 * jj + 1)
                return 0
            lax.fori_loop(0, 128, _inner, 0)
            return 0
        lax.fori_loop(0, P // 4096, _outer, 0)

        def _src(q, _):
            def _sv(j, _):
                k = aux_v[pl.ds(q * 2048 + j * 16, 16)]
                node = lo + q * 2048 + j * 16 + iota16
                srcb[pl.ds(j * 16, 16)] = jnp.where(k >= 0, k, node + P)
                return 0
            lax.fori_loop(0, 128, _sv, 0)
            pltpu.sync_copy(srcb, src_out.at[pl.ds(lo + q * 2048, 2048)])
            return 0
        lax.fori_loop(0, R // 2048, _src, 0)


def _sc_emit(src_hbm, comb_hbm, out_hbm, srcv, rbuf, sem):
    wid = lax.axis_index("s") * NC + lax.axis_index("c")
    n0 = wid * (N // NW)

    def _chunk(ch, _):
        base = n0 + ch * 512
        pltpu.sync_copy(src_hbm.at[pl.ds(base, 512)], srcv)

        def _g(k, _):
            pltpu.async_copy(
                comb_hbm.at[srcv.at[pl.ds(k * 128, 128)]],
                rbuf.at[pl.ds(k * 128, 128)], sem).wait()
            return 0
        lax.fori_loop(0, 4, _g, 0)
        pltpu.sync_copy(rbuf, out_hbm.at[pl.ds(base, 512)])
        return 0
    lax.fori_loop(0, N // NW // 512, _chunk, 0)


def _sc_mesh():
    return plsc.VectorSubcoreMesh(core_axis_name="c", subcore_axis_name="s",
                                  num_cores=NC, num_subcores=NS)


def _fused_call(*args):
    return pl.kernel(
        _sc_fused,
        out_type=(jax.ShapeDtypeStruct((P,), _i32),
                  jax.ShapeDtypeStruct((P,), _i32),
                  jax.ShapeDtypeStruct((P_ID, D), _f32),
                  jax.ShapeDtypeStruct((N,), _i32)),
        scratch_types=[
            pltpu.VMEM((8192,), _i32),
            pltpu.VMEM((4096,), _i32),
            pltpu.VMEM((8192,), _i32),
            pltpu.VMEM((8192,), _i32),
            pltpu.VMEM((512, D), _f32),
            pltpu.VMEM((N // 16,), _i32),
            pltpu.VMEM((4096,), _i32),
            pltpu.VMEM((2048,), _i32),
            pltpu.VMEM((16,), _i32),
            pltpu.SemaphoreType.DMA,
        ],
        mesh=_sc_mesh(),
        compiler_params=pltpu.CompilerParams(needs_layout_passes=False),
    )(*args)


def _emit_call(*args):
    return pl.kernel(
        _sc_emit,
        out_type=jax.ShapeDtypeStruct((N, D), _f32),
        scratch_types=[
            pltpu.VMEM((512,), _i32),
            pltpu.VMEM((512, D), _f32),
            pltpu.SemaphoreType.DMA,
        ],
        mesh=_sc_mesh(),
        compiler_params=pltpu.CompilerParams(needs_layout_passes=False),
    )(*args)


def _fold(table, w_part):
    return jnp.dot(table, w_part, preferred_element_type=_f32)


def _fold_stage(type_emb, major_emb, minor_emb, nrc_emb, pos_emb, w):
    a, b, c = w[0:128], w[128:160], w[160:192]
    t1 = _fold(type_emb, a)
    t2 = jnp.pad(_fold(major_emb, a[0:85]), ((0, 2), (0, 0)))
    t3 = jnp.pad(_fold(minor_emb, a[85:128]), ((0, 4), (0, 0)))
    t4 = _fold(nrc_emb, b)
    t5 = _fold(pos_emb, c)
    return t1, t2, t3, t4, t5


def kernel(ast_node_types, ast_node_major_types, ast_node_minor_types,
           ast_node_nr_children, ast_node_child_ltr_position,
           ast_node_child_rtl_position, id_leaf_node_indices,
           id_leaf_identifier_idx, prim_leaf_node_indices,
           prim_leaf_primitive_type, mod_leaf_node_indices, mod_leaf_modifier,
           identifiers_encodings, type_emb, major_emb, minor_emb, nrc_emb,
           pos_emb, prim_emb, mod_emb, W_id, b_id, W_prim, b_prim, W_mod,
           b_mod, W_wo, b_wo):
    ii = lambda x: x.astype(_i32)
    w0 = (ii(ast_node_types) | (ii(ast_node_major_types) << 8)
          | (ii(ast_node_minor_types) << 13) | (ii(ast_node_nr_children) << 19))
    w1 = ii(ast_node_child_ltr_position) | (ii(ast_node_child_rtl_position) << 6)
    leaf_nodes = jnp.concatenate([ii(id_leaf_node_indices),
                                  ii(prim_leaf_node_indices),
                                  ii(mod_leaf_node_indices)])
    featpm = jnp.concatenate([ii(prim_leaf_primitive_type),
                              ii(mod_leaf_modifier)]).reshape(-1, 1)

    tw = _fold_stage(type_emb, major_emb, minor_emb, nrc_emb, pos_emb, W_wo)
    tid = _fold_stage(type_emb, major_emb, minor_emb, nrc_emb, pos_emb,
                      W_id[D:])
    tpr = _fold_stage(type_emb, major_emb, minor_emb, nrc_emb, pos_emb,
                      W_prim[64:])
    tmo = _fold_stage(type_emb, major_emb, minor_emb, nrc_emb, pos_emb,
                      W_mod[64:])
    stk = [jnp.stack([tid[j], tpr[j], tmo[j]]) for j in range(5)]
    ft_stk = jnp.stack([jnp.zeros((16, D), _f32),
                        _fold(prim_emb, W_prim[0:64]),
                        _fold(mod_emb, W_mod[0:64])])
    b_stk = jnp.stack([b_id.reshape(1, D), b_prim.reshape(1, D),
                       b_mod.reshape(1, D)])
    w_ida = W_id[0:D]

    full = lambda s: pl.BlockSpec(s, lambda i: (0,) * len(s))
    combined0 = pl.pallas_call(
        _base_body,
        grid=(N // TILE,),
        in_specs=[
            pl.BlockSpec((TILE, 1), lambda i: (i, 0)),
            pl.BlockSpec((TILE, 1), lambda i: (i, 0)),
            full((200, D)), full((32, D)), full((64, D)), full((32, D)),
            full((64, D)), full((1, D)),
        ],
        out_specs=pl.BlockSpec((TILE, D), lambda i: (i + P // TILE, 0)),
        out_shape=jax.ShapeDtypeStruct((C, D), _f32),
    )(w0.reshape(N, 1), w1.reshape(N, 1), *tw, b_wo.reshape(1, D))

    w0_leaf, w1_leaf, idrows, src = _fused_call(
        w0, w1, leaf_nodes, identifiers_encodings,
        ii(id_leaf_identifier_idx))

    nid = P_ID // TILE
    r_of = lambda i: jnp.where(i < nid, 0,
                               jnp.where(i < nid + P_PT // TILE, 1, 2))
    combined = pl.pallas_call(
        _leaf_body,
        grid=(P // TILE,),
        in_specs=[
            pl.BlockSpec(memory_space=pl.ANY),
            pl.BlockSpec((TILE, 1), lambda i: (i, 0)),
            pl.BlockSpec((TILE, 1), lambda i: (i, 0)),
            pl.BlockSpec((TILE, D), lambda i: (jnp.minimum(i, nid - 1), 0)),
            pl.BlockSpec((TILE, 1),
                         lambda i: (jnp.clip(i - nid, 0, nid - 1), 0)),
            full((D, D)),
            pl.BlockSpec((1, 200, D), lambda i: (r_of(i), 0, 0)),
            pl.BlockSpec((1, 32, D), lambda i: (r_of(i), 0, 0)),
            pl.BlockSpec((1, 64, D), lambda i: (r_of(i), 0, 0)),
            pl.BlockSpec((1, 32, D), lambda i: (r_of(i), 0, 0)),
            pl.BlockSpec((1, 64, D), lambda i: (r_of(i), 0, 0)),
            pl.BlockSpec((1, 16, D), lambda i: (r_of(i), 0, 0)),
            pl.BlockSpec((1, 1, D), lambda i: (r_of(i), 0, 0)),
        ],
        out_specs=pl.BlockSpec((TILE, D), lambda i: (i, 0)),
        out_shape=jax.ShapeDtypeStruct((C, D), _f32),
        input_output_aliases={0: 0},
    )(combined0, w0_leaf.reshape(P, 1), w1_leaf.reshape(P, 1), idrows,
      featpm, w_ida, *stk, ft_stk, b_stk)

    return _emit_call(src, combined)


# speedup vs baseline: 17.9854x; 1.1216x over previous
import jax
import jax.numpy as jnp
from jax import lax
from jax.experimental import pallas as pl
from jax.experimental.pallas import tpu as pltpu
from jax.experimental.pallas import tpu_sc as plsc

N = 262144
P_ID = 65536
P_PT = 32768
P_MOD = 32768
P = P_ID + P_PT + P_MOD
C = P + N
D = 128
PAD = 0

TILE = 2048
NC = 2
NS = 16
NW = NC * NS

_f32 = jnp.float32
_i32 = jnp.int32


def _oh(idx_col, v, dtype=_f32):
    rows = idx_col.shape[0]
    io = lax.broadcasted_iota(_i32, (rows, v), 1)
    return (idx_col == io).astype(dtype)


def _unpack(w0, w1):
    typ = w0 & 255
    maj = (w0 >> 8) & 31
    mnr = (w0 >> 13) & 63
    nrc = (w0 >> 19) & 31
    ltr = w1 & 63
    rtl = (w1 >> 6) & 63
    return typ, maj, mnr, nrc, ltr, rtl


def _base_body(w0_ref, w1_ref, t1_ref, t2_ref, t3_ref, t4_ref, t5_ref, b_ref,
               out_ref):
    typ, maj, mnr, nrc, ltr, rtl = _unpack(w0_ref[...], w1_ref[...])
    first = jnp.where(
        mnr == PAD,
        jnp.dot(_oh(typ, 200), t1_ref[...], preferred_element_type=_f32),
        jnp.dot(_oh(maj, 32), t2_ref[...], preferred_element_type=_f32)
        + jnp.dot(_oh(mnr, 64), t3_ref[...], preferred_element_type=_f32),
    )
    acc = (first
           + jnp.dot(_oh(nrc, 32), t4_ref[...], preferred_element_type=_f32)
           + jnp.dot(_oh(ltr, 64) + _oh(rtl, 64), t5_ref[...],
                     preferred_element_type=_f32)
           + b_ref[...])
    out_ref[...] = jnp.maximum(acc, 0.0)


def _leaf_body(comb_in_ref, w0_ref, w1_ref, idenc_ref, featpm_ref, wida_ref,
               t1_ref, t2_ref, t3_ref, t4_ref, t5_ref, ft_ref, b_ref,
               out_ref):
    del comb_in_ref
    i = pl.program_id(0)
    typ, maj, mnr, nrc, ltr, rtl = _unpack(w0_ref[...], w1_ref[...])
    first = jnp.where(
        mnr == PAD,
        jnp.dot(_oh(typ, 200), t1_ref[0], preferred_element_type=_f32),
        jnp.dot(_oh(maj, 32), t2_ref[0], preferred_element_type=_f32)
        + jnp.dot(_oh(mnr, 64), t3_ref[0], preferred_element_type=_f32),
    )
    acc = (first
           + jnp.dot(_oh(nrc, 32), t4_ref[0], preferred_element_type=_f32)
           + jnp.dot(_oh(ltr, 64) + _oh(rtl, 64), t5_ref[0],
                     preferred_element_type=_f32)
           + b_ref[0])

    @pl.when(i < P_ID // TILE)
    def _id_region():
        feat = jnp.dot(idenc_ref[...], wida_ref[...],
                       preferred_element_type=_f32)
        out_ref[...] = jnp.maximum(acc + feat, 0.0)

    @pl.when(i >= P_ID // TILE)
    def _pm_region():
        feat = jnp.dot(_oh(featpm_ref[...], 16), ft_ref[0],
                       preferred_element_type=_f32)
        out_ref[...] = jnp.maximum(acc + feat, 0.0)


def _sc_fused(w0_hbm, w1_hbm, lnodes_hbm, ident_hbm, ididx_hbm,
              w0_out, w1_out, idrows_out, src_out,
              idxa_v, idxi_v, b0, b1, rbuf, aux_v, sbuf, srcb, tmp16, sem):
    wid = lax.axis_index("s") * NC + lax.axis_index("c")
    iota16 = lax.iota(_i32, 16)
    nxtidx = jnp.minimum(iota16 + 1, 15)

    @pl.when(wid < 16)
    def _gather_half():
        g = wid
        a0 = g * (P // 16)
        pltpu.sync_copy(lnodes_hbm.at[pl.ds(a0, 8192)], idxa_v)

        def _ga(k, _):
            sl = pl.ds(k * 128, 128)
            pltpu.async_copy(w0_hbm.at[idxa_v.at[sl]], b0.at[sl], sem).wait()
            pltpu.async_copy(w1_hbm.at[idxa_v.at[sl]], b1.at[sl], sem).wait()
            return 0
        lax.fori_loop(0, 64, _ga, 0)
        pltpu.sync_copy(b0, w0_out.at[pl.ds(a0, 8192)])
        pltpu.sync_copy(b1, w1_out.at[pl.ds(a0, 8192)])
        i0 = g * (P_ID // 16)
        pltpu.sync_copy(ididx_hbm.at[pl.ds(i0, 4096)], idxi_v)
        for oc in range(8):
            def _gi(k, _, oc=oc):
                pltpu.async_copy(
                    ident_hbm.at[idxi_v.at[pl.ds(oc * 512 + k * 128, 128)]],
                    rbuf.at[pl.ds(k * 128, 128)], sem).wait()
                return 0
            lax.fori_loop(0, 4, _gi, 0)
            pltpu.sync_copy(rbuf, idrows_out.at[pl.ds(i0 + oc * 512, 512)])

    @pl.when(wid >= 16)
    def _winner_half():
        R = N // 16
        lo = (wid - 16) * R
        neg16 = jnp.full((16,), -1, _i32)

        def _z(j, _):
            aux_v[pl.ds(j * 16, 16)] = neg16
            return 0
        lax.fori_loop(0, R // 16, _z, 0)

        def _one(cb, j):
            v = sbuf[pl.ds(j * 16, 16)]
            m = (v >= lo) & (v < lo + R)

            @pl.when(jnp.any(m))
            def _hit():
                pos = cb * 4096 + j * 16 + iota16
                c = jnp.where(m, ((v - lo) << 17) | pos, -1)
                sc, _unused = plsc.sort_key_val(c, c)
                tmp16[...] = sc
                nxt = plsc.load_gather(tmp16, [nxtidx])
                islast = ((sc >> 17) != (nxt >> 17)) | (iota16 == 15)
                upd = islast & (sc >= 0)
                addr = jnp.maximum(sc >> 17, 0)
                plsc.store_scatter(aux_v, [addr], sc & 0x1FFFF, mask=upd)

        def _outer(cb, _):
            pltpu.sync_copy(lnodes_hbm.at[pl.ds(cb * 4096, 4096)], sbuf)

            def _inner(jj, _):
                _one(cb, 2 * jj)
                _one(cb, 2 * jj + 1)
                return 0
            lax.fori_loop(0, 128, _inner, 0)
            return 0
        lax.fori_loop(0, P // 4096, _outer, 0)

        def _src(q, _):
            def _sv(j, _):
                k = aux_v[pl.ds(q * 2048 + j * 16, 16)]
                node = lo + q * 2048 + j * 16 + iota16
                srcb[pl.ds(j * 16, 16)] = jnp.where(k >= 0, k, node + P)
                return 0
            lax.fori_loop(0, 128, _sv, 0)
            pltpu.sync_copy(srcb, src_out.at[pl.ds(lo + q * 2048, 2048)])
            return 0
        lax.fori_loop(0, R // 2048, _src, 0)


def _sc_emit(src_hbm, comb_hbm, out_hbm, srcv, rbuf, sem):
    wid = lax.axis_index("s") * NC + lax.axis_index("c")
    n0 = wid * (N // NW)

    def _chunk(ch, _):
        base = n0 + ch * 512
        pltpu.sync_copy(src_hbm.at[pl.ds(base, 512)], srcv)

        def _g(k, _):
            pltpu.async_copy(
                comb_hbm.at[srcv.at[pl.ds(k * 128, 128)]],
                rbuf.at[pl.ds(k * 128, 128)], sem).wait()
            return 0
        lax.fori_loop(0, 4, _g, 0)
        pltpu.sync_copy(rbuf, out_hbm.at[pl.ds(base, 512)])
        return 0
    lax.fori_loop(0, N // NW // 512, _chunk, 0)


def _sc_mesh():
    return plsc.VectorSubcoreMesh(core_axis_name="c", subcore_axis_name="s",
                                  num_cores=NC, num_subcores=NS)


def _fused_call(*args):
    return pl.kernel(
        _sc_fused,
        out_type=(jax.ShapeDtypeStruct((P,), _i32),
                  jax.ShapeDtypeStruct((P,), _i32),
                  jax.ShapeDtypeStruct((P_ID, D), _f32),
                  jax.ShapeDtypeStruct((N,), _i32)),
        scratch_types=[
            pltpu.VMEM((8192,), _i32),
            pltpu.VMEM((4096,), _i32),
            pltpu.VMEM((8192,), _i32),
            pltpu.VMEM((8192,), _i32),
            pltpu.VMEM((512, D), _f32),
            pltpu.VMEM((N // 16,), _i32),
            pltpu.VMEM((4096,), _i32),
            pltpu.VMEM((2048,), _i32),
            pltpu.VMEM((16,), _i32),
            pltpu.SemaphoreType.DMA,
        ],
        mesh=_sc_mesh(),
        compiler_params=pltpu.CompilerParams(needs_layout_passes=False),
    )(*args)


def _emit_call(*args):
    return pl.kernel(
        _sc_emit,
        out_type=jax.ShapeDtypeStruct((N, D), _f32),
        scratch_types=[
            pltpu.VMEM((512,), _i32),
            pltpu.VMEM((512, D), _f32),
            pltpu.SemaphoreType.DMA,
        ],
        mesh=_sc_mesh(),
        compiler_params=pltpu.CompilerParams(needs_layout_passes=False),
    )(*args)


def _fold(table, w_part):
    return jnp.dot(table, w_part, preferred_element_type=_f32)


def _fold_stage(type_emb, major_emb, minor_emb, nrc_emb, pos_emb, w):
    a, b, c = w[0:128], w[128:160], w[160:192]
    t1 = _fold(type_emb, a)
    t2 = jnp.pad(_fold(major_emb, a[0:85]), ((0, 2), (0, 0)))
    t3 = jnp.pad(_fold(minor_emb, a[85:128]), ((0, 4), (0, 0)))
    t4 = _fold(nrc_emb, b)
    t5 = _fold(pos_emb, c)
    return t1, t2, t3, t4, t5


def kernel(ast_node_types, ast_node_major_types, ast_node_minor_types,
           ast_node_nr_children, ast_node_child_ltr_position,
           ast_node_child_rtl_position, id_leaf_node_indices,
           id_leaf_identifier_idx, prim_leaf_node_indices,
           prim_leaf_primitive_type, mod_leaf_node_indices, mod_leaf_modifier,
           identifiers_encodings, type_emb, major_emb, minor_emb, nrc_emb,
           pos_emb, prim_emb, mod_emb, W_id, b_id, W_prim, b_prim, W_mod,
           b_mod, W_wo, b_wo):
    ii = lambda x: x.astype(_i32)
    w0 = (ii(ast_node_types) | (ii(ast_node_major_types) << 8)
          | (ii(ast_node_minor_types) << 13) | (ii(ast_node_nr_children) << 19))
    w1 = ii(ast_node_child_ltr_position) | (ii(ast_node_child_rtl_position) << 6)
    leaf_nodes = jnp.concatenate([ii(id_leaf_node_indices),
                                  ii(prim_leaf_node_indices),
                                  ii(mod_leaf_node_indices)])
    featpm = jnp.concatenate([ii(prim_leaf_primitive_type),
                              ii(mod_leaf_modifier)]).reshape(-1, 1)

    tw = _fold_stage(type_emb, major_emb, minor_emb, nrc_emb, pos_emb, W_wo)
    tid = _fold_stage(type_emb, major_emb, minor_emb, nrc_emb, pos_emb,
                      W_id[D:])
    tpr = _fold_stage(type_emb, major_emb, minor_emb, nrc_emb, pos_emb,
                      W_prim[64:])
    tmo = _fold_stage(type_emb, major_emb, minor_emb, nrc_emb, pos_emb,
                      W_mod[64:])
    stk = [jnp.stack([tid[j], tpr[j], tmo[j]]) for j in range(5)]
    ft_stk = jnp.stack([jnp.zeros((16, D), _f32),
                        _fold(prim_emb, W_prim[0:64]),
                        _fold(mod_emb, W_mod[0:64])])
    b_stk = jnp.stack([b_id.reshape(1, D), b_prim.reshape(1, D),
                       b_mod.reshape(1, D)])
    w_ida = W_id[0:D]

    full = lambda s: pl.BlockSpec(s, lambda i: (0,) * len(s))
    combined0 = pl.pallas_call(
        _base_body,
        grid=(N // TILE,),
        in_specs=[
            pl.BlockSpec((TILE, 1), lambda i: (i, 0)),
            pl.BlockSpec((TILE, 1), lambda i: (i, 0)),
            full((200, D)), full((32, D)), full((64, D)), full((32, D)),
            full((64, D)), full((1, D)),
        ],
        out_specs=pl.BlockSpec((TILE, D), lambda i: (i + P // TILE, 0)),
        out_shape=jax.ShapeDtypeStruct((C, D), _f32),
    )(w0.reshape(N, 1), w1.reshape(N, 1), *tw, b_wo.reshape(1, D))

    w0_leaf, w1_leaf, idrows, src = _fused_call(
        w0, w1, leaf_nodes, identifiers_encodings,
        ii(id_leaf_identifier_idx))

    nid = P_ID // TILE
    r_of = lambda i: jnp.where(i < nid, 0,
                               jnp.where(i < nid + P_PT // TILE, 1, 2))
    combined = pl.pallas_call(
        _leaf_body,
        grid=(P // TILE,),
        in_specs=[
            pl.BlockSpec(memory_space=pl.ANY),
            pl.BlockSpec((TILE, 1), lambda i: (i, 0)),
            pl.BlockSpec((TILE, 1), lambda i: (i, 0)),
            pl.BlockSpec((TILE, D), lambda i: (jnp.minimum(i, nid - 1), 0)),
            pl.BlockSpec((TILE, 1),
                         lambda i: (jnp.clip(i - nid, 0, nid - 1), 0)),
            full((D, D)),
            pl.BlockSpec((1, 200, D), lambda i: (r_of(i), 0, 0)),
            pl.BlockSpec((1, 32, D), lambda i: (r_of(i), 0, 0)),
            pl.BlockSpec((1, 64, D), lambda i: (r_of(i), 0, 0)),
            pl.BlockSpec((1, 32, D), lambda i: (r_of(i), 0, 0)),
            pl.BlockSpec((1, 64, D), lambda i: (r_of(i), 0, 0)),
            pl.BlockSpec((1, 16, D), lambda i: (r_of(i), 0, 0)),
            pl.BlockSpec((1, 1, D), lambda i: (r_of(i), 0, 0)),
        ],
        out_specs=pl.BlockSpec((TILE, D), lambda i: (i, 0)),
        out_shape=jax.ShapeDtypeStruct((C, D), _f32),
        input_output_aliases={0: 0},
    )(combined0, w0_leaf.reshape(P, 1), w1_leaf.reshape(P, 1), idrows,
      featpm, w_ida, *stk, ft_stk, b_stk)

    return _emit_call(src, combined)
